# Initial kernel scaffold; baseline (speedup 1.0000x reference)
import jax
import jax.numpy as jnp
from jax.experimental import pallas as pl


def kernel(x_constraint, x_operator, edge_index_co, edge_index_oc, batch_constraint, batch_operator, params):
    raise NotImplementedError("write your pallas kernel here")


# speedup vs baseline: 3.0870x; 3.0870x over previous
import functools

import jax
import jax.numpy as jnp
from jax import lax
from jax.experimental import pallas as pl
from jax.experimental.pallas import tpu as pltpu
from jax.experimental.pallas import tpu_sc as plsc

_N = 10000
_NP = 10240
_E = 320000
_H = 128
_NG = 64
_RB = 512
_GRID = _NP // _RB
_TILES = 16
_CHUNK = 128
_GC = 16
_NGRP = 10
_NCH = _GC * _NGRP
_EPT = _NCH * _CHUNK
_EPAD = _EPT * _TILES
_ZROWS = 16
_RPT = _NP // _TILES


def _make_sc_kernel(with_counts):
  out_type = [
      jax.ShapeDtypeStruct((_NP, _H), jnp.float32),
      jax.ShapeDtypeStruct((_NP, _H), jnp.float32),
  ]
  if with_counts:
    out_type += [
        jax.ShapeDtypeStruct((_NP,), jnp.float32),
        jax.ShapeDtypeStruct((_NP,), jnp.float32),
    ]
  scratch = [
      pltpu.VMEM((_GC, _CHUNK), jnp.int32),
      pltpu.VMEM((_GC, _CHUNK), jnp.int32),
      pltpu.VMEM((_CHUNK, _H), jnp.float32),
      pltpu.VMEM((_ZROWS, _H), jnp.float32),
      pltpu.VMEM_SHARED((_NP, _H), jnp.float32),
      pltpu.SemaphoreType.DMA,
  ]
  if with_counts:
    scratch += [
        pltpu.VMEM((_CHUNK,), jnp.float32),
        pltpu.VMEM((_RPT,), jnp.float32),
        pltpu.VMEM_SHARED((_NP,), jnp.float32),
    ]
  mesh = plsc.VectorSubcoreMesh(core_axis_name="c", subcore_axis_name="s")

  def body(table_o, table_c, oc_src, oc_dst, co_src, co_dst, *rest):
    if with_counts:
      (sum_c, sum_o, cnt_c, cnt_o,
       idx_s, idx_d, rows, zrow, acc, sem, ones_v, zcnt, cacc) = rest
    else:
      (sum_c, sum_o, idx_s, idx_d, rows, zrow, acc, sem) = rest
      cnt_c = cnt_o = ones_v = zcnt = cacc = None

    cid = lax.axis_index("c")
    sid = lax.axis_index("s")
    zv = jnp.zeros((16,), jnp.float32)

    def zr(i, _):
      zrow[i // 8, pl.ds((i % 8) * 16, 16)] = zv
      return 0
    lax.fori_loop(0, _ZROWS * 8, zr, 0)

    def za(k, _):
      pltpu.sync_copy(zrow, acc.at[pl.ds(sid * _RPT + k * _ZROWS, _ZROWS)])
      return 0
    lax.fori_loop(0, _RPT // _ZROWS, za, 0)

    if with_counts:
      def zc(i, _):
        zcnt[pl.ds(i * 16, 16)] = zv
        return 0
      lax.fori_loop(0, _RPT // 16, zc, 0)
      ov = jnp.ones((16,), jnp.float32)
      def ob(i, _):
        ones_v[pl.ds(i * 16, 16)] = ov
        return 0
      lax.fori_loop(0, _CHUNK // 16, ob, 0)
      pltpu.sync_copy(zcnt, cacc.at[pl.ds(sid * _RPT, _RPT)])

    def run_dir(tbl, src_h, dst_h, out_h, cnt_h):
      plsc.subcore_barrier()

      def group(g, _):
        pltpu.sync_copy(src_h.at[sid, pl.ds(g * _GC, _GC)], idx_s)
        pltpu.sync_copy(dst_h.at[sid, pl.ds(g * _GC, _GC)], idx_d)

        def chunk(j, _):
          pltpu.async_copy(tbl.at[idx_s.at[j]], rows, sem).wait()
          pltpu.sync_copy(rows, acc.at[idx_d.at[j]], add=True)
          if with_counts:
            pltpu.sync_copy(ones_v, cacc.at[idx_d.at[j]], add=True)
          return 0
        lax.fori_loop(0, _GC, chunk, 0)
        return 0
      lax.fori_loop(0, _NGRP, group, 0)

      plsc.subcore_barrier()
      pltpu.sync_copy(acc.at[pl.ds(sid * _RPT, _RPT)],
                      out_h.at[pl.ds(sid * _RPT, _RPT)])
      if with_counts:
        pltpu.sync_copy(cacc.at[pl.ds(sid * _RPT, _RPT)],
                        cnt_h.at[pl.ds(sid * _RPT, _RPT)])

    pl.when(cid == 0)(lambda: run_dir(table_o, oc_src, oc_dst, sum_c, cnt_c))
    pl.when(cid == 1)(lambda: run_dir(table_c, co_src, co_dst, sum_o, cnt_o))

  return pl.kernel(body, out_type=out_type, mesh=mesh, scratch_types=scratch)


@functools.cache
def _get_sc_kernel(with_counts):
  return _make_sc_kernel(with_counts)


def _dot(a, b):
  return jnp.dot(a, b, preferred_element_type=jnp.float32)


def _full(shape):
  return pl.BlockSpec(shape, lambda i: (0, 0))


def _blk():
  return pl.BlockSpec((_RB, _H), lambda i: (i, 0))


def _k1_body(xc, xo, wlc, blc, wlo, blo, wmc, wmo, oxc, oxo, otc, oto):
  a = _dot(xc[...], wlc[...]) + blc[...]
  b = _dot(xo[...], wlo[...]) + blo[...]
  oxc[...] = a
  oxo[...] = b
  otc[...] = _dot(a, wmc[...])
  oto[...] = _dot(b, wmo[...])


def _k1(xc, xo, wlc, blc, wlo, blo, wmc, wmo):
  f = jax.ShapeDtypeStruct((_NP, _H), jnp.float32)
  return pl.pallas_call(
      _k1_body,
      grid=(_GRID,),
      in_specs=[_blk(), _blk(), _full((_H, _H)), _full((1, _H)),
                _full((_H, _H)), _full((1, _H)), _full((_H, _H)),
                _full((_H, _H))],
      out_specs=[_blk(), _blk(), _blk(), _blk()],
      out_shape=[f, f, f, f],
  )(xc, xo, wlc, blc, wlo, blo, wmc, wmo)


def _sigmoid(v):
  return 1.0 / (1.0 + jnp.exp(-v))


def _gru_update(x, s, cnt, wself, bconv, wi, bi, bh):
  m = s / jnp.maximum(cnt, 1.0)
  h = jnp.maximum(_dot(x, wself) + m + bconv, 0.0)
  gi = _dot(h, wi) + bi
  i_r = gi[:, :_H]
  i_z = gi[:, _H:2 * _H]
  i_n = gi[:, 2 * _H:]
  r = _sigmoid(i_r + bh[:, :_H])
  z = _sigmoid(i_z + bh[:, _H:2 * _H])
  n = jnp.tanh(i_n + r * bh[:, 2 * _H:])
  return (1.0 - z) * n


def _k3_body(xc, xo, sc, so, cc, co,
             wsc, bcc, wic, bic, bhc, wmc,
             wso, bco, wio, bio, bho, wmo,
             oxc, oxo, otc, oto):
  a = _gru_update(xc[...], sc[...], cc[...], wsc[...], bcc[...], wic[...],
                  bic[...], bhc[...])
  b = _gru_update(xo[...], so[...], co[...], wso[...], bco[...], wio[...],
                  bio[...], bho[...])
  oxc[...] = a
  oxo[...] = b
  otc[...] = _dot(a, wmc[...])
  oto[...] = _dot(b, wmo[...])


def _k3(xc, xo, sc, so, cc, co, wc, wo):
  f = jax.ShapeDtypeStruct((_NP, _H), jnp.float32)
  wspec = [_full((_H, _H)), _full((1, _H)), _full((_H, 3 * _H)),
           _full((1, 3 * _H)), _full((1, 3 * _H)), _full((_H, _H))]
  cspec = pl.BlockSpec((_RB, 1), lambda i: (i, 0))
  return pl.pallas_call(
      _k3_body,
      grid=(_GRID,),
      in_specs=[_blk(), _blk(), _blk(), _blk(), cspec, cspec] + wspec + wspec,
      out_specs=[_blk(), _blk(), _blk(), _blk()],
      out_shape=[f, f, f, f],
  )(xc, xo, sc, so, cc, co, *wc, *wo)


def _k5_body(xc, xo, sc, so, cc, co, bidc, bido,
             wsc, bcc, wic, bic, bhc,
             wso, bco, wio, bio, bho,
             psc, pcc, pso, pco):
  i = pl.program_id(0)
  a = _gru_update(xc[...], sc[...], cc[...], wsc[...], bcc[...], wic[...],
                  bic[...], bhc[...])
  b = _gru_update(xo[...], so[...], co[...], wso[...], bco[...], wio[...],
                  bio[...], bho[...])
  gids = lax.broadcasted_iota(jnp.int32, (_RB, _NG), 1)
  ohc = (bidc[...] == gids).astype(jnp.float32)
  oho = (bido[...] == gids).astype(jnp.float32)
  ones = jnp.ones((_RB, _H), jnp.float32)
  dn = (((0,), (0,)), ((), ()))

  @pl.when(i == 0)
  def _():
    psc[...] = jnp.zeros_like(psc)
    pcc[...] = jnp.zeros_like(pcc)
    pso[...] = jnp.zeros_like(pso)
    pco[...] = jnp.zeros_like(pco)

  psc[...] += lax.dot_general(ohc, a, dn, preferred_element_type=jnp.float32)
  pcc[...] += lax.dot_general(ohc, ones, dn,
                              preferred_element_type=jnp.float32)
  pso[...] += lax.dot_general(oho, b, dn, preferred_element_type=jnp.float32)
  pco[...] += lax.dot_general(oho, ones, dn,
                              preferred_element_type=jnp.float32)


def _k5(xc, xo, sc, so, cc, co, bidc, bido, wc, wo):
  g = jax.ShapeDtypeStruct((_NG, _H), jnp.float32)
  wspec = [_full((_H, _H)), _full((1, _H)), _full((_H, 3 * _H)),
           _full((1, 3 * _H)), _full((1, 3 * _H))]
  cspec = pl.BlockSpec((_RB, 1), lambda i: (i, 0))
  gspec = pl.BlockSpec((_NG, _H), lambda i: (0, 0))
  return pl.pallas_call(
      _k5_body,
      grid=(_GRID,),
      in_specs=[_blk(), _blk(), _blk(), _blk(), cspec, cspec, cspec, cspec]
      + wspec + wspec,
      out_specs=[gspec, gspec, gspec, gspec],
      out_shape=[g, g, g, g],
  )(xc, xo, sc, so, cc, co, bidc, bido, *wc, *wo)


def _k6_body(psc, pcc, pso, pco, linw, linb, outw, outb, out):
  a = psc[...] / jnp.maximum(pcc[...], 1.0)
  b = pso[...] / jnp.maximum(pco[...], 1.0)
  x = jnp.concatenate([a, b], axis=1)
  x = jnp.maximum(_dot(x, linw[...]) + linb[...], 0.0)
  x = jnp.maximum(_dot(x, linw[...]) + linb[...], 0.0)
  out[...] = _dot(x, outw[...]) + outb[...]


def _k6(psc, pcc, pso, pco, linw, linb, outw, outb):
  return pl.pallas_call(
      _k6_body,
      out_shape=jax.ShapeDtypeStruct((_NG, _H), jnp.float32),
  )(psc, pcc, pso, pco, linw, linb, outw, outb)


def _prep_edges(ei):
  src = jnp.pad(ei[0].astype(jnp.int32), (0, _EPAD - _E))
  dst = jnp.pad(ei[1].astype(jnp.int32), (0, _EPAD - _E), constant_values=_N)
  return (src.reshape(_TILES, _NCH, _CHUNK),
          dst.reshape(_TILES, _NCH, _CHUNK))


def kernel(x_constraint, x_operator, edge_index_co, edge_index_oc,
           batch_constraint, batch_operator, params):
  p = params
  xc = jnp.pad(x_constraint, ((0, _NP - _N), (0, 0)))
  xo = jnp.pad(x_operator, ((0, _NP - _N), (0, 0)))
  oc_s, oc_d = _prep_edges(edge_index_oc)
  co_s, co_d = _prep_edges(edge_index_co)
  bidc = jnp.pad(batch_constraint.astype(jnp.int32), (0, _NP - _N),
                 constant_values=_NG).reshape(_NP, 1)
  bido = jnp.pad(batch_operator.astype(jnp.int32), (0, _NP - _N),
                 constant_values=_NG).reshape(_NP, 1)

  def r1(v):
    return v.reshape(1, -1)

  xc0, xo0, tc0, to0 = _k1(
      xc, xo, p['lin_W_constraint'], r1(p['lin_b_constraint']),
      p['lin_W_operator'], r1(p['lin_b_operator']),
      p['W_msg_constraint_0'], p['W_msg_operator_0'])

  sum_c0, sum_o0, cnt_c, cnt_o = _get_sc_kernel(True)(
      to0, tc0, oc_s, oc_d, co_s, co_d)
  cc = cnt_c.reshape(_NP, 1)
  co = cnt_o.reshape(_NP, 1)

  def wpack(t, l, with_msg_next):
    w = [p['W_self_%s_%d' % (t, l)], r1(p['b_conv_%s_%d' % (t, l)]),
         p['gru_Wi_' + t], r1(p['gru_bi_' + t]), r1(p['gru_bh_' + t])]
    if with_msg_next:
      w.append(p['W_msg_%s_%d' % (t, l + 1)])
    return w

  xc1, xo1, tc1, to1 = _k3(xc0, xo0, sum_c0, sum_o0, cc, co,
                           wpack('constraint', 0, True),
                           wpack('operator', 0, True))

  sum_c1, sum_o1 = _get_sc_kernel(False)(to1, tc1, oc_s, oc_d, co_s, co_d)

  psc, pcc, pso, pco = _k5(xc1, xo1, sum_c1, sum_o1, cc, co, bidc, bido,
                           wpack('constraint', 1, False),
                           wpack('operator', 1, False))

  outw = jnp.pad(p['out_W'], ((0, 0), (0, _H - p['out_W'].shape[1])))
  outb = jnp.pad(p['out_b'], (0, _H - p['out_b'].shape[0])).reshape(1, _H)
  out = _k6(psc, pcc, pso, pco, p['lin_W'], r1(p['lin_b']), outw, outb)
  return out[:, :p['out_W'].shape[1]]


# speedup vs baseline: 3.5664x; 1.1553x over previous
import functools

import jax
import jax.numpy as jnp
from jax import lax
from jax.experimental import pallas as pl
from jax.experimental.pallas import tpu as pltpu
from jax.experimental.pallas import tpu_sc as plsc

_N = 10000
_NP = 10240
_E = 320000
_H = 128
_NG = 64
_RB = 512
_GRID = _NP // _RB
_TILES = 16
_CHUNK = 128
_GC = 8
_NGRP = 20
_NCH = _GC * _NGRP
_EPT = _NCH * _CHUNK
_EPAD = _EPT * _TILES
_ZROWS = 16
_RPT = _NP // _TILES


def _make_sc_kernel(with_counts):
  out_type = [
      jax.ShapeDtypeStruct((_NP, _H), jnp.float32),
      jax.ShapeDtypeStruct((_NP, _H), jnp.float32),
  ]
  if with_counts:
    out_type += [
        jax.ShapeDtypeStruct((_NP,), jnp.float32),
        jax.ShapeDtypeStruct((_NP,), jnp.float32),
    ]
  scratch = [
      pltpu.VMEM((_GC, _CHUNK), jnp.int32),
      pltpu.VMEM((_GC, _CHUNK), jnp.int32),
      pltpu.VMEM((_CHUNK, _H), jnp.float32),
      pltpu.VMEM((_CHUNK, _H), jnp.float32),
      pltpu.VMEM((_ZROWS, _H), jnp.float32),
      pltpu.VMEM_SHARED((_NP, _H), jnp.float32),
      pltpu.SemaphoreType.DMA,
      pltpu.SemaphoreType.DMA,
  ]
  if with_counts:
    scratch += [
        pltpu.VMEM((_CHUNK,), jnp.float32),
        pltpu.VMEM((_RPT,), jnp.float32),
        pltpu.VMEM_SHARED((_NP,), jnp.float32),
    ]
  mesh = plsc.VectorSubcoreMesh(core_axis_name="c", subcore_axis_name="s")

  def body(table_o, table_c, oc_src, oc_dst, co_src, co_dst, *rest):
    if with_counts:
      (sum_c, sum_o, cnt_c, cnt_o,
       idx_s, idx_d, rows0, rows1, zrow, acc, sem0, sem1,
       ones_v, zcnt, cacc) = rest
    else:
      (sum_c, sum_o, idx_s, idx_d, rows0, rows1, zrow, acc,
       sem0, sem1) = rest
      cnt_c = cnt_o = ones_v = zcnt = cacc = None
    rbufs = (rows0, rows1)
    sems = (sem0, sem1)

    cid = lax.axis_index("c")
    sid = lax.axis_index("s")
    zv = jnp.zeros((16,), jnp.float32)

    def zr(i, _):
      zrow[i // 8, pl.ds((i % 8) * 16, 16)] = zv
      return 0
    lax.fori_loop(0, _ZROWS * 8, zr, 0)

    def za(k, _):
      pltpu.sync_copy(zrow, acc.at[pl.ds(sid * _RPT + k * _ZROWS, _ZROWS)])
      return 0
    lax.fori_loop(0, _RPT // _ZROWS, za, 0)

    if with_counts:
      def zc(i, _):
        zcnt[pl.ds(i * 16, 16)] = zv
        return 0
      lax.fori_loop(0, _RPT // 16, zc, 0)
      ov = jnp.ones((16,), jnp.float32)
      def ob(i, _):
        ones_v[pl.ds(i * 16, 16)] = ov
        return 0
      lax.fori_loop(0, _CHUNK // 16, ob, 0)
      pltpu.sync_copy(zcnt, cacc.at[pl.ds(sid * _RPT, _RPT)])

    def run_dir(tbl, src_h, dst_h, out_h, cnt_h):
      plsc.subcore_barrier()

      def group(g, _):
        pltpu.sync_copy(src_h.at[sid, pl.ds(g * _GC, _GC)], idx_s)
        pltpu.sync_copy(dst_h.at[sid, pl.ds(g * _GC, _GC)], idx_d)
        descs = [None, None]
        descs[0] = pltpu.async_copy(tbl.at[idx_s.at[0]], rbufs[0], sems[0])
        for j in range(_GC):
          b = j % 2
          if j + 1 < _GC:
            nb = (j + 1) % 2
            descs[nb] = pltpu.async_copy(
                tbl.at[idx_s.at[j + 1]], rbufs[nb], sems[nb])
          descs[b].wait()
          pltpu.sync_copy(rbufs[b], acc.at[idx_d.at[j]], add=True)
          if with_counts:
            pltpu.sync_copy(ones_v, cacc.at[idx_d.at[j]], add=True)
        return 0
      lax.fori_loop(0, _NGRP, group, 0)

      plsc.subcore_barrier()
      pltpu.sync_copy(acc.at[pl.ds(sid * _RPT, _RPT)],
                      out_h.at[pl.ds(sid * _RPT, _RPT)])
      if with_counts:
        pltpu.sync_copy(cacc.at[pl.ds(sid * _RPT, _RPT)],
                        cnt_h.at[pl.ds(sid * _RPT, _RPT)])

    pl.when(cid == 0)(lambda: run_dir(table_o, oc_src, oc_dst, sum_c, cnt_c))
    pl.when(cid == 1)(lambda: run_dir(table_c, co_src, co_dst, sum_o, cnt_o))

  return pl.kernel(body, out_type=out_type, mesh=mesh, scratch_types=scratch)


@functools.cache
def _get_sc_kernel(with_counts):
  return _make_sc_kernel(with_counts)


def _dot(a, b):
  return jnp.dot(a, b, preferred_element_type=jnp.float32)


def _full(shape):
  return pl.BlockSpec(shape, lambda i: (0, 0))


def _blk():
  return pl.BlockSpec((_RB, _H), lambda i: (i, 0))


def _k1_body(xc, xo, wlc, blc, wlo, blo, wmc, wmo, oxc, oxo, otc, oto):
  a = _dot(xc[...], wlc[...]) + blc[...]
  b = _dot(xo[...], wlo[...]) + blo[...]
  oxc[...] = a
  oxo[...] = b
  otc[...] = _dot(a, wmc[...])
  oto[...] = _dot(b, wmo[...])


def _k1(xc, xo, wlc, blc, wlo, blo, wmc, wmo):
  f = jax.ShapeDtypeStruct((_NP, _H), jnp.float32)
  return pl.pallas_call(
      _k1_body,
      grid=(_GRID,),
      in_specs=[_blk(), _blk(), _full((_H, _H)), _full((1, _H)),
                _full((_H, _H)), _full((1, _H)), _full((_H, _H)),
                _full((_H, _H))],
      out_specs=[_blk(), _blk(), _blk(), _blk()],
      out_shape=[f, f, f, f],
  )(xc, xo, wlc, blc, wlo, blo, wmc, wmo)


def _sigmoid(v):
  return 1.0 / (1.0 + jnp.exp(-v))


def _gru_update(x, s, cnt, wself, bconv, wi, bi, bh):
  m = s / jnp.maximum(cnt, 1.0)
  h = jnp.maximum(_dot(x, wself) + m + bconv, 0.0)
  gi = _dot(h, wi) + bi
  i_r = gi[:, :_H]
  i_z = gi[:, _H:2 * _H]
  i_n = gi[:, 2 * _H:]
  r = _sigmoid(i_r + bh[:, :_H])
  z = _sigmoid(i_z + bh[:, _H:2 * _H])
  n = jnp.tanh(i_n + r * bh[:, 2 * _H:])
  return (1.0 - z) * n


def _k3_body(xc, xo, sc, so, cc, co,
             wsc, bcc, wic, bic, bhc, wmc,
             wso, bco, wio, bio, bho, wmo,
             oxc, oxo, otc, oto):
  a = _gru_update(xc[...], sc[...], cc[...], wsc[...], bcc[...], wic[...],
                  bic[...], bhc[...])
  b = _gru_update(xo[...], so[...], co[...], wso[...], bco[...], wio[...],
                  bio[...], bho[...])
  oxc[...] = a
  oxo[...] = b
  otc[...] = _dot(a, wmc[...])
  oto[...] = _dot(b, wmo[...])


def _k3(xc, xo, sc, so, cc, co, wc, wo):
  f = jax.ShapeDtypeStruct((_NP, _H), jnp.float32)
  wspec = [_full((_H, _H)), _full((1, _H)), _full((_H, 3 * _H)),
           _full((1, 3 * _H)), _full((1, 3 * _H)), _full((_H, _H))]
  cspec = pl.BlockSpec((_RB, 1), lambda i: (i, 0))
  return pl.pallas_call(
      _k3_body,
      grid=(_GRID,),
      in_specs=[_blk(), _blk(), _blk(), _blk(), cspec, cspec] + wspec + wspec,
      out_specs=[_blk(), _blk(), _blk(), _blk()],
      out_shape=[f, f, f, f],
  )(xc, xo, sc, so, cc, co, *wc, *wo)


def _k5_body(xc, xo, sc, so, cc, co, bidc, bido,
             wsc, bcc, wic, bic, bhc,
             wso, bco, wio, bio, bho,
             psc, pcc, pso, pco):
  i = pl.program_id(0)
  a = _gru_update(xc[...], sc[...], cc[...], wsc[...], bcc[...], wic[...],
                  bic[...], bhc[...])
  b = _gru_update(xo[...], so[...], co[...], wso[...], bco[...], wio[...],
                  bio[...], bho[...])
  gids = lax.broadcasted_iota(jnp.int32, (_RB, _NG), 1)
  ohc = (bidc[...] == gids).astype(jnp.float32)
  oho = (bido[...] == gids).astype(jnp.float32)
  ones = jnp.ones((_RB, _H), jnp.float32)
  dn = (((0,), (0,)), ((), ()))

  @pl.when(i == 0)
  def _():
    psc[...] = jnp.zeros_like(psc)
    pcc[...] = jnp.zeros_like(pcc)
    pso[...] = jnp.zeros_like(pso)
    pco[...] = jnp.zeros_like(pco)

  psc[...] += lax.dot_general(ohc, a, dn, preferred_element_type=jnp.float32)
  pcc[...] += lax.dot_general(ohc, ones, dn,
                              preferred_element_type=jnp.float32)
  pso[...] += lax.dot_general(oho, b, dn, preferred_element_type=jnp.float32)
  pco[...] += lax.dot_general(oho, ones, dn,
                              preferred_element_type=jnp.float32)


def _k5(xc, xo, sc, so, cc, co, bidc, bido, wc, wo):
  g = jax.ShapeDtypeStruct((_NG, _H), jnp.float32)
  wspec = [_full((_H, _H)), _full((1, _H)), _full((_H, 3 * _H)),
           _full((1, 3 * _H)), _full((1, 3 * _H))]
  cspec = pl.BlockSpec((_RB, 1), lambda i: (i, 0))
  gspec = pl.BlockSpec((_NG, _H), lambda i: (0, 0))
  return pl.pallas_call(
      _k5_body,
      grid=(_GRID,),
      in_specs=[_blk(), _blk(), _blk(), _blk(), cspec, cspec, cspec, cspec]
      + wspec + wspec,
      out_specs=[gspec, gspec, gspec, gspec],
      out_shape=[g, g, g, g],
  )(xc, xo, sc, so, cc, co, bidc, bido, *wc, *wo)


def _k6_body(psc, pcc, pso, pco, linw, linb, outw, outb, out):
  a = psc[...] / jnp.maximum(pcc[...], 1.0)
  b = pso[...] / jnp.maximum(pco[...], 1.0)
  x = jnp.concatenate([a, b], axis=1)
  x = jnp.maximum(_dot(x, linw[...]) + linb[...], 0.0)
  x = jnp.maximum(_dot(x, linw[...]) + linb[...], 0.0)
  out[...] = _dot(x, outw[...]) + outb[...]


def _k6(psc, pcc, pso, pco, linw, linb, outw, outb):
  return pl.pallas_call(
      _k6_body,
      out_shape=jax.ShapeDtypeStruct((_NG, _H), jnp.float32),
  )(psc, pcc, pso, pco, linw, linb, outw, outb)


def _prep_edges(ei):
  src = jnp.pad(ei[0].astype(jnp.int32), (0, _EPAD - _E))
  dst = jnp.pad(ei[1].astype(jnp.int32), (0, _EPAD - _E), constant_values=_N)
  return (src.reshape(_TILES, _NCH, _CHUNK),
          dst.reshape(_TILES, _NCH, _CHUNK))


def kernel(x_constraint, x_operator, edge_index_co, edge_index_oc,
           batch_constraint, batch_operator, params):
  p = params
  xc = jnp.pad(x_constraint, ((0, _NP - _N), (0, 0)))
  xo = jnp.pad(x_operator, ((0, _NP - _N), (0, 0)))
  oc_s, oc_d = _prep_edges(edge_index_oc)
  co_s, co_d = _prep_edges(edge_index_co)
  bidc = jnp.pad(batch_constraint.astype(jnp.int32), (0, _NP - _N),
                 constant_values=_NG).reshape(_NP, 1)
  bido = jnp.pad(batch_operator.astype(jnp.int32), (0, _NP - _N),
                 constant_values=_NG).reshape(_NP, 1)

  def r1(v):
    return v.reshape(1, -1)

  xc0, xo0, tc0, to0 = _k1(
      xc, xo, p['lin_W_constraint'], r1(p['lin_b_constraint']),
      p['lin_W_operator'], r1(p['lin_b_operator']),
      p['W_msg_constraint_0'], p['W_msg_operator_0'])

  sum_c0, sum_o0, cnt_c, cnt_o = _get_sc_kernel(True)(
      to0, tc0, oc_s, oc_d, co_s, co_d)
  cc = cnt_c.reshape(_NP, 1)
  co = cnt_o.reshape(_NP, 1)

  def wpack(t, l, with_msg_next):
    w = [p['W_self_%s_%d' % (t, l)], r1(p['b_conv_%s_%d' % (t, l)]),
         p['gru_Wi_' + t], r1(p['gru_bi_' + t]), r1(p['gru_bh_' + t])]
    if with_msg_next:
      w.append(p['W_msg_%s_%d' % (t, l + 1)])
    return w

  xc1, xo1, tc1, to1 = _k3(xc0, xo0, sum_c0, sum_o0, cc, co,
                           wpack('constraint', 0, True),
                           wpack('operator', 0, True))

  sum_c1, sum_o1 = _get_sc_kernel(False)(to1, tc1, oc_s, oc_d, co_s, co_d)

  psc, pcc, pso, pco = _k5(xc1, xo1, sum_c1, sum_o1, cc, co, bidc, bido,
                           wpack('constraint', 1, False),
                           wpack('operator', 1, False))

  outw = jnp.pad(p['out_W'], ((0, 0), (0, _H - p['out_W'].shape[1])))
  outb = jnp.pad(p['out_b'], (0, _H - p['out_b'].shape[0])).reshape(1, _H)
  out = _k6(psc, pcc, pso, pco, p['lin_W'], r1(p['lin_b']), outw, outb)
  return out[:, :p['out_W'].shape[1]]


# speedup vs baseline: 3.5804x; 1.0039x over previous
import functools

import jax
import jax.numpy as jnp
from jax import lax
from jax.experimental import pallas as pl
from jax.experimental.pallas import tpu as pltpu
from jax.experimental.pallas import tpu_sc as plsc

_N = 10000
_NP = 10240
_E = 320000
_H = 128
_NG = 64
_RB = 512
_GRID = _NP // _RB
_TILES = 16
_CHUNK = 128
_GC = 8
_NGRP = 20
_NCH = _GC * _NGRP
_EPT = _NCH * _CHUNK
_EPAD = _EPT * _TILES
_ZROWS = 16
_RPT = _NP // _TILES


def _make_sc_kernel(with_counts):
  out_type = [
      jax.ShapeDtypeStruct((_NP, _H), jnp.float32),
      jax.ShapeDtypeStruct((_NP, _H), jnp.float32),
  ]
  if with_counts:
    out_type += [
        jax.ShapeDtypeStruct((_NP,), jnp.float32),
        jax.ShapeDtypeStruct((_NP,), jnp.float32),
    ]
  scratch = [
      pltpu.VMEM((_GC, _CHUNK), jnp.int32),
      pltpu.VMEM((_GC, _CHUNK), jnp.int32),
      pltpu.VMEM((_CHUNK, _H), jnp.float32),
      pltpu.VMEM((_CHUNK, _H), jnp.float32),
      pltpu.VMEM((_ZROWS, _H), jnp.float32),
      pltpu.VMEM_SHARED((_NP, _H), jnp.float32),
      pltpu.SemaphoreType.DMA,
      pltpu.SemaphoreType.DMA,
      pltpu.SemaphoreType.DMA,
      pltpu.SemaphoreType.DMA,
      pltpu.SemaphoreType.DMA,
  ]
  if with_counts:
    scratch += [
        pltpu.VMEM((_CHUNK,), jnp.float32),
        pltpu.VMEM((_RPT,), jnp.float32),
        pltpu.VMEM_SHARED((_NP,), jnp.float32),
    ]
  mesh = plsc.VectorSubcoreMesh(core_axis_name="c", subcore_axis_name="s")

  def body(table_o, table_c, oc_src, oc_dst, co_src, co_dst, *rest):
    if with_counts:
      (sum_c, sum_o, cnt_c, cnt_o,
       idx_s, idx_d, rows0, rows1, zrow, acc, sg0, sg1, ss0, ss1, sct,
       ones_v, zcnt, cacc) = rest
    else:
      (sum_c, sum_o, idx_s, idx_d, rows0, rows1, zrow, acc,
       sg0, sg1, ss0, ss1, sct) = rest
      cnt_c = cnt_o = ones_v = zcnt = cacc = None
    rbufs = (rows0, rows1)
    gsems = (sg0, sg1)
    ssems = (ss0, ss1)

    cid = lax.axis_index("c")
    sid = lax.axis_index("s")
    zv = jnp.zeros((16,), jnp.float32)

    def zr(i, _):
      zrow[i // 8, pl.ds((i % 8) * 16, 16)] = zv
      return 0
    lax.fori_loop(0, _ZROWS * 8, zr, 0)

    def za(k, _):
      pltpu.sync_copy(zrow, acc.at[pl.ds(sid * _RPT + k * _ZROWS, _ZROWS)])
      return 0
    lax.fori_loop(0, _RPT // _ZROWS, za, 0)

    if with_counts:
      def zc(i, _):
        zcnt[pl.ds(i * 16, 16)] = zv
        return 0
      lax.fori_loop(0, _RPT // 16, zc, 0)
      ov = jnp.ones((16,), jnp.float32)
      def ob(i, _):
        ones_v[pl.ds(i * 16, 16)] = ov
        return 0
      lax.fori_loop(0, _CHUNK // 16, ob, 0)
      pltpu.sync_copy(zcnt, cacc.at[pl.ds(sid * _RPT, _RPT)])

    def run_dir(tbl, src_h, dst_h, out_h, cnt_h):
      plsc.subcore_barrier()

      def group(g, _):
        pltpu.sync_copy(src_h.at[sid, pl.ds(g * _GC, _GC)], idx_s)
        pltpu.sync_copy(dst_h.at[sid, pl.ds(g * _GC, _GC)], idx_d)
        gdescs = [None, None]
        sdescs = [None, None]
        cdescs = []
        gdescs[0] = pltpu.async_copy(tbl.at[idx_s.at[0]], rbufs[0], gsems[0])
        for j in range(_GC):
          b = j % 2
          if j + 1 < _GC:
            nb = (j + 1) % 2
            if sdescs[nb] is not None:
              sdescs[nb].wait()
              sdescs[nb] = None
            gdescs[nb] = pltpu.async_copy(
                tbl.at[idx_s.at[j + 1]], rbufs[nb], gsems[nb])
          gdescs[b].wait()
          sdescs[b] = pltpu.async_copy(
              rbufs[b], acc.at[idx_d.at[j]], ssems[b], add=True)
          if with_counts:
            cdescs.append(pltpu.async_copy(
                ones_v, cacc.at[idx_d.at[j]], sct, add=True))
        for d in sdescs:
          if d is not None:
            d.wait()
        for d in cdescs:
          d.wait()
        return 0
      lax.fori_loop(0, _NGRP, group, 0)

      plsc.subcore_barrier()
      pltpu.sync_copy(acc.at[pl.ds(sid * _RPT, _RPT)],
                      out_h.at[pl.ds(sid * _RPT, _RPT)])
      if with_counts:
        pltpu.sync_copy(cacc.at[pl.ds(sid * _RPT, _RPT)],
                        cnt_h.at[pl.ds(sid * _RPT, _RPT)])

    pl.when(cid == 0)(lambda: run_dir(table_o, oc_src, oc_dst, sum_c, cnt_c))
    pl.when(cid == 1)(lambda: run_dir(table_c, co_src, co_dst, sum_o, cnt_o))

  return pl.kernel(body, out_type=out_type, mesh=mesh, scratch_types=scratch)


@functools.cache
def _get_sc_kernel(with_counts):
  return _make_sc_kernel(with_counts)


def _dot(a, b):
  return jnp.dot(a, b, preferred_element_type=jnp.float32)


def _full(shape):
  return pl.BlockSpec(shape, lambda i: (0, 0))


def _blk():
  return pl.BlockSpec((_RB, _H), lambda i: (i, 0))


def _k1_body(xc, xo, wlc, blc, wlo, blo, wmc, wmo, oxc, oxo, otc, oto):
  a = _dot(xc[...], wlc[...]) + blc[...]
  b = _dot(xo[...], wlo[...]) + blo[...]
  oxc[...] = a
  oxo[...] = b
  otc[...] = _dot(a, wmc[...])
  oto[...] = _dot(b, wmo[...])


def _k1(xc, xo, wlc, blc, wlo, blo, wmc, wmo):
  f = jax.ShapeDtypeStruct((_NP, _H), jnp.float32)
  return pl.pallas_call(
      _k1_body,
      grid=(_GRID,),
      in_specs=[_blk(), _blk(), _full((_H, _H)), _full((1, _H)),
                _full((_H, _H)), _full((1, _H)), _full((_H, _H)),
                _full((_H, _H))],
      out_specs=[_blk(), _blk(), _blk(), _blk()],
      out_shape=[f, f, f, f],
  )(xc, xo, wlc, blc, wlo, blo, wmc, wmo)


def _sigmoid(v):
  return 1.0 / (1.0 + jnp.exp(-v))


def _gru_update(x, s, cnt, wself, bconv, wi, bi, bh):
  m = s / jnp.maximum(cnt, 1.0)
  h = jnp.maximum(_dot(x, wself) + m + bconv, 0.0)
  gi = _dot(h, wi) + bi
  i_r = gi[:, :_H]
  i_z = gi[:, _H:2 * _H]
  i_n = gi[:, 2 * _H:]
  r = _sigmoid(i_r + bh[:, :_H])
  z = _sigmoid(i_z + bh[:, _H:2 * _H])
  n = jnp.tanh(i_n + r * bh[:, 2 * _H:])
  return (1.0 - z) * n


def _k3_body(xc, xo, sc, so, cc, co,
             wsc, bcc, wic, bic, bhc, wmc,
             wso, bco, wio, bio, bho, wmo,
             oxc, oxo, otc, oto):
  a = _gru_update(xc[...], sc[...], cc[...], wsc[...], bcc[...], wic[...],
                  bic[...], bhc[...])
  b = _gru_update(xo[...], so[...], co[...], wso[...], bco[...], wio[...],
                  bio[...], bho[...])
  oxc[...] = a
  oxo[...] = b
  otc[...] = _dot(a, wmc[...])
  oto[...] = _dot(b, wmo[...])


def _k3(xc, xo, sc, so, cc, co, wc, wo):
  f = jax.ShapeDtypeStruct((_NP, _H), jnp.float32)
  wspec = [_full((_H, _H)), _full((1, _H)), _full((_H, 3 * _H)),
           _full((1, 3 * _H)), _full((1, 3 * _H)), _full((_H, _H))]
  cspec = pl.BlockSpec((_RB, 1), lambda i: (i, 0))
  return pl.pallas_call(
      _k3_body,
      grid=(_GRID,),
      in_specs=[_blk(), _blk(), _blk(), _blk(), cspec, cspec] + wspec + wspec,
      out_specs=[_blk(), _blk(), _blk(), _blk()],
      out_shape=[f, f, f, f],
  )(xc, xo, sc, so, cc, co, *wc, *wo)


def _k5_body(xc, xo, sc, so, cc, co, bidc, bido,
             wsc, bcc, wic, bic, bhc,
             wso, bco, wio, bio, bho,
             psc, pcc, pso, pco):
  i = pl.program_id(0)
  a = _gru_update(xc[...], sc[...], cc[...], wsc[...], bcc[...], wic[...],
                  bic[...], bhc[...])
  b = _gru_update(xo[...], so[...], co[...], wso[...], bco[...], wio[...],
                  bio[...], bho[...])
  gids = lax.broadcasted_iota(jnp.int32, (_RB, _NG), 1)
  ohc = (bidc[...] == gids).astype(jnp.float32)
  oho = (bido[...] == gids).astype(jnp.float32)
  ones = jnp.ones((_RB, _H), jnp.float32)
  dn = (((0,), (0,)), ((), ()))

  @pl.when(i == 0)
  def _():
    psc[...] = jnp.zeros_like(psc)
    pcc[...] = jnp.zeros_like(pcc)
    pso[...] = jnp.zeros_like(pso)
    pco[...] = jnp.zeros_like(pco)

  psc[...] += lax.dot_general(ohc, a, dn, preferred_element_type=jnp.float32)
  pcc[...] += lax.dot_general(ohc, ones, dn,
                              preferred_element_type=jnp.float32)
  pso[...] += lax.dot_general(oho, b, dn, preferred_element_type=jnp.float32)
  pco[...] += lax.dot_general(oho, ones, dn,
                              preferred_element_type=jnp.float32)


def _k5(xc, xo, sc, so, cc, co, bidc, bido, wc, wo):
  g = jax.ShapeDtypeStruct((_NG, _H), jnp.float32)
  wspec = [_full((_H, _H)), _full((1, _H)), _full((_H, 3 * _H)),
           _full((1, 3 * _H)), _full((1, 3 * _H))]
  cspec = pl.BlockSpec((_RB, 1), lambda i: (i, 0))
  gspec = pl.BlockSpec((_NG, _H), lambda i: (0, 0))
  return pl.pallas_call(
      _k5_body,
      grid=(_GRID,),
      in_specs=[_blk(), _blk(), _blk(), _blk(), cspec, cspec, cspec, cspec]
      + wspec + wspec,
      out_specs=[gspec, gspec, gspec, gspec],
      out_shape=[g, g, g, g],
  )(xc, xo, sc, so, cc, co, bidc, bido, *wc, *wo)


def _k6_body(psc, pcc, pso, pco, linw, linb, outw, outb, out):
  a = psc[...] / jnp.maximum(pcc[...], 1.0)
  b = pso[...] / jnp.maximum(pco[...], 1.0)
  x = jnp.concatenate([a, b], axis=1)
  x = jnp.maximum(_dot(x, linw[...]) + linb[...], 0.0)
  x = jnp.maximum(_dot(x, linw[...]) + linb[...], 0.0)
  out[...] = _dot(x, outw[...]) + outb[...]


def _k6(psc, pcc, pso, pco, linw, linb, outw, outb):
  return pl.pallas_call(
      _k6_body,
      out_shape=jax.ShapeDtypeStruct((_NG, _H), jnp.float32),
  )(psc, pcc, pso, pco, linw, linb, outw, outb)


def _prep_edges(ei):
  src = jnp.pad(ei[0].astype(jnp.int32), (0, _EPAD - _E))
  dst = jnp.pad(ei[1].astype(jnp.int32), (0, _EPAD - _E), constant_values=_N)
  return (src.reshape(_TILES, _NCH, _CHUNK),
          dst.reshape(_TILES, _NCH, _CHUNK))


def kernel(x_constraint, x_operator, edge_index_co, edge_index_oc,
           batch_constraint, batch_operator, params):
  p = params
  xc = jnp.pad(x_constraint, ((0, _NP - _N), (0, 0)))
  xo = jnp.pad(x_operator, ((0, _NP - _N), (0, 0)))
  oc_s, oc_d = _prep_edges(edge_index_oc)
  co_s, co_d = _prep_edges(edge_index_co)
  bidc = jnp.pad(batch_constraint.astype(jnp.int32), (0, _NP - _N),
                 constant_values=_NG).reshape(_NP, 1)
  bido = jnp.pad(batch_operator.astype(jnp.int32), (0, _NP - _N),
                 constant_values=_NG).reshape(_NP, 1)

  def r1(v):
    return v.reshape(1, -1)

  xc0, xo0, tc0, to0 = _k1(
      xc, xo, p['lin_W_constraint'], r1(p['lin_b_constraint']),
      p['lin_W_operator'], r1(p['lin_b_operator']),
      p['W_msg_constraint_0'], p['W_msg_operator_0'])

  sum_c0, sum_o0, cnt_c, cnt_o = _get_sc_kernel(True)(
      to0, tc0, oc_s, oc_d, co_s, co_d)
  cc = cnt_c.reshape(_NP, 1)
  co = cnt_o.reshape(_NP, 1)

  def wpack(t, l, with_msg_next):
    w = [p['W_self_%s_%d' % (t, l)], r1(p['b_conv_%s_%d' % (t, l)]),
         p['gru_Wi_' + t], r1(p['gru_bi_' + t]), r1(p['gru_bh_' + t])]
    if with_msg_next:
      w.append(p['W_msg_%s_%d' % (t, l + 1)])
    return w

  xc1, xo1, tc1, to1 = _k3(xc0, xo0, sum_c0, sum_o0, cc, co,
                           wpack('constraint', 0, True),
                           wpack('operator', 0, True))

  sum_c1, sum_o1 = _get_sc_kernel(False)(to1, tc1, oc_s, oc_d, co_s, co_d)

  psc, pcc, pso, pco = _k5(xc1, xo1, sum_c1, sum_o1, cc, co, bidc, bido,
                           wpack('constraint', 1, False),
                           wpack('operator', 1, False))

  outw = jnp.pad(p['out_W'], ((0, 0), (0, _H - p['out_W'].shape[1])))
  outb = jnp.pad(p['out_b'], (0, _H - p['out_b'].shape[0])).reshape(1, _H)
  out = _k6(psc, pcc, pso, pco, p['lin_W'], r1(p['lin_b']), outw, outb)
  return out[:, :p['out_W'].shape[1]]


# speedup vs baseline: 3.6298x; 1.0138x over previous
import functools

import jax
import jax.numpy as jnp
from jax import lax
from jax.experimental import pallas as pl
from jax.experimental.pallas import tpu as pltpu
from jax.experimental.pallas import tpu_sc as plsc

_N = 10000
_NP = 10240
_E = 320000
_H = 128
_NG = 64
_RB = 1024
_GRID = _NP // _RB
_TILES = 16
_CHUNK = 128
_GC = 8
_NGRP = 20
_NR = _NP // _H
_NCH = _GC * _NGRP
_EPT = _NCH * _CHUNK
_EPAD = _EPT * _TILES
_ZROWS = 16
_RPT = _NP // _TILES


def _make_sc_kernel(with_counts):
  out_type = [
      jax.ShapeDtypeStruct((_NP, _H), jnp.float32),
      jax.ShapeDtypeStruct((_NP, _H), jnp.float32),
  ]
  if with_counts:
    out_type += [
        jax.ShapeDtypeStruct((_NP,), jnp.float32),
        jax.ShapeDtypeStruct((_NP,), jnp.float32),
    ]
  scratch = [
      pltpu.VMEM((_GC, _CHUNK), jnp.int32),
      pltpu.VMEM((_GC, _CHUNK), jnp.int32),
      pltpu.VMEM((_CHUNK, _H), jnp.float32),
      pltpu.VMEM((_CHUNK, _H), jnp.float32),
      pltpu.VMEM((_ZROWS, _H), jnp.float32),
      pltpu.VMEM_SHARED((_NP, _H), jnp.float32),
      pltpu.SemaphoreType.DMA,
      pltpu.SemaphoreType.DMA,
      pltpu.SemaphoreType.DMA,
      pltpu.SemaphoreType.DMA,
  ]
  if with_counts:
    scratch += [
        pltpu.SemaphoreType.DMA,
        pltpu.VMEM((_CHUNK,), jnp.float32),
        pltpu.VMEM((_RPT,), jnp.float32),
        pltpu.VMEM_SHARED((_NP,), jnp.float32),
    ]
  mesh = plsc.VectorSubcoreMesh(core_axis_name="c", subcore_axis_name="s")

  def body(table_o, table_c, oc_src, oc_dst, co_src, co_dst, *rest):
    if with_counts:
      (sum_c, sum_o, cnt_c, cnt_o,
       idx_s, idx_d, rows0, rows1, zrow, acc, sg0, sg1, ss0, ss1,
       sct, ones_v, zcnt, cacc) = rest
    else:
      (sum_c, sum_o, idx_s, idx_d, rows0, rows1, zrow, acc,
       sg0, sg1, ss0, ss1) = rest
      cnt_c = cnt_o = sct = ones_v = zcnt = cacc = None
    rbufs = (rows0, rows1)
    gsems = (sg0, sg1)
    ssems = (ss0, ss1)

    cid = lax.axis_index("c")
    sid = lax.axis_index("s")
    zv = jnp.zeros((16,), jnp.float32)

    def zr(i, _):
      zrow[i // 8, pl.ds((i % 8) * 16, 16)] = zv
      return 0
    lax.fori_loop(0, _ZROWS * 8, zr, 0)

    def za(k, _):
      pltpu.sync_copy(zrow, acc.at[pl.ds(sid * _RPT + k * _ZROWS, _ZROWS)])
      return 0
    lax.fori_loop(0, _RPT // _ZROWS, za, 0)

    if with_counts:
      def zc(i, _):
        zcnt[pl.ds(i * 16, 16)] = zv
        return 0
      lax.fori_loop(0, _RPT // 16, zc, 0)
      ov = jnp.ones((16,), jnp.float32)
      def ob(i, _):
        ones_v[pl.ds(i * 16, 16)] = ov
        return 0
      lax.fori_loop(0, _CHUNK // 16, ob, 0)
      pltpu.sync_copy(zcnt, cacc.at[pl.ds(sid * _RPT, _RPT)])

    def run_dir(tbl, src_h, dst_h, out_h, cnt_h):
      plsc.subcore_barrier()

      def group(g, _):
        pltpu.sync_copy(src_h.at[sid, pl.ds(g * _GC, _GC)], idx_s)
        pltpu.sync_copy(dst_h.at[sid, pl.ds(g * _GC, _GC)], idx_d)
        gdescs = [None, None]
        sdescs = [None, None]
        cdescs = []
        gdescs[0] = pltpu.async_copy(tbl.at[idx_s.at[0]], rbufs[0], gsems[0])
        for j in range(_GC):
          b = j % 2
          if j + 1 < _GC:
            nb = (j + 1) % 2
            if sdescs[nb] is not None:
              sdescs[nb].wait()
              sdescs[nb] = None
            gdescs[nb] = pltpu.async_copy(
                tbl.at[idx_s.at[j + 1]], rbufs[nb], gsems[nb])
          gdescs[b].wait()
          sdescs[b] = pltpu.async_copy(
              rbufs[b], acc.at[idx_d.at[j]], ssems[b], add=True)
          if with_counts:
            cdescs.append(pltpu.async_copy(
                ones_v, cacc.at[idx_d.at[j]], sct, add=True))
        for d in sdescs:
          if d is not None:
            d.wait()
        for d in cdescs:
          d.wait()
        return 0
      lax.fori_loop(0, _NGRP, group, 0)

      plsc.subcore_barrier()
      pltpu.sync_copy(acc.at[pl.ds(sid * _RPT, _RPT)],
                      out_h.at[pl.ds(sid * _RPT, _RPT)])
      if with_counts:
        pltpu.sync_copy(cacc.at[pl.ds(sid * _RPT, _RPT)],
                        cnt_h.at[pl.ds(sid * _RPT, _RPT)])

    pl.when(cid == 0)(lambda: run_dir(table_o, oc_src, oc_dst, sum_c, cnt_c))
    pl.when(cid == 1)(lambda: run_dir(table_c, co_src, co_dst, sum_o, cnt_o))

  return pl.kernel(body, out_type=out_type, mesh=mesh, scratch_types=scratch)


@functools.cache
def _get_sc_kernel(with_counts):
  return _make_sc_kernel(with_counts)


def _dot(a, b):
  return jnp.dot(a, b, preferred_element_type=jnp.float32)


def _full(shape):
  return pl.BlockSpec(shape, lambda i: (0, 0))


def _blk():
  return pl.BlockSpec((_RB, _H), lambda i: (i, 0))


def _k1_body(xc, xo, wlc, blc, wlo, blo, wmc, wmo, oxc, oxo, otc, oto):
  a = _dot(xc[...], wlc[...]) + blc[...]
  b = _dot(xo[...], wlo[...]) + blo[...]
  oxc[...] = a
  oxo[...] = b
  otc[...] = _dot(a, wmc[...])
  oto[...] = _dot(b, wmo[...])


def _k1(xc, xo, wlc, blc, wlo, blo, wmc, wmo):
  f = jax.ShapeDtypeStruct((_NP, _H), jnp.float32)
  return pl.pallas_call(
      _k1_body,
      grid=(_GRID,),
      in_specs=[_blk(), _blk(), _full((_H, _H)), _full((1, _H)),
                _full((_H, _H)), _full((1, _H)), _full((_H, _H)),
                _full((_H, _H))],
      out_specs=[_blk(), _blk(), _blk(), _blk()],
      out_shape=[f, f, f, f],
  )(xc, xo, wlc, blc, wlo, blo, wmc, wmo)


def _sigmoid(v):
  return 1.0 / (1.0 + jnp.exp(-v))


def _gru_update(x, s, cnt, wself, bconv, wi, bi, bh):
  m = s / jnp.maximum(cnt, 1.0)
  h = jnp.maximum(_dot(x, wself) + m + bconv, 0.0)
  gi = _dot(h, wi) + bi
  i_r = gi[:, :_H]
  i_z = gi[:, _H:2 * _H]
  i_n = gi[:, 2 * _H:]
  r = _sigmoid(i_r + bh[:, :_H])
  z = _sigmoid(i_z + bh[:, _H:2 * _H])
  n = jnp.tanh(i_n + r * bh[:, 2 * _H:])
  return (1.0 - z) * n


def _k3_body(xc, xo, sc, so, cc, co,
             wsc, bcc, wic, bic, bhc, wmc,
             wso, bco, wio, bio, bho, wmo,
             oxc, oxo, otc, oto):
  a = _gru_update(xc[...], sc[...], cc[...], wsc[...], bcc[...],
                  wic[...], bic[...], bhc[...])
  b = _gru_update(xo[...], so[...], co[...], wso[...], bco[...],
                  wio[...], bio[...], bho[...])
  oxc[...] = a
  oxo[...] = b
  otc[...] = _dot(a, wmc[...])
  oto[...] = _dot(b, wmo[...])


def _k3(xc, xo, sc, so, cc, co, wc, wo):
  f = jax.ShapeDtypeStruct((_NP, _H), jnp.float32)
  wspec = [_full((_H, _H)), _full((1, _H)), _full((_H, 3 * _H)),
           _full((1, 3 * _H)), _full((1, 3 * _H)), _full((_H, _H))]
  cspec = pl.BlockSpec((_RB, 1), lambda i: (i, 0))
  return pl.pallas_call(
      _k3_body,
      grid=(_GRID,),
      in_specs=[_blk(), _blk(), _blk(), _blk(), cspec, cspec] + wspec + wspec,
      out_specs=[_blk(), _blk(), _blk(), _blk()],
      out_shape=[f, f, f, f],
  )(xc, xo, sc, so, cc, co, *wc, *wo)


def _k5_body(xc, xo, sc, so, cc, co, bidc, bido,
             wsc, bcc, wic, bic, bhc,
             wso, bco, wio, bio, bho,
             psc, pcc, pso, pco):
  i = pl.program_id(0)
  a = _gru_update(xc[...], sc[...], cc[...], wsc[...], bcc[...],
                  wic[...], bic[...], bhc[...])
  b = _gru_update(xo[...], so[...], co[...], wso[...], bco[...],
                  wio[...], bio[...], bho[...])
  gids = lax.broadcasted_iota(jnp.int32, (_RB, _NG), 1)
  ohc = (bidc[...] == gids).astype(jnp.float32)
  oho = (bido[...] == gids).astype(jnp.float32)
  ones = jnp.ones((_RB, _H), jnp.float32)
  dn = (((0,), (0,)), ((), ()))

  @pl.when(i == 0)
  def _():
    psc[...] = jnp.zeros_like(psc)
    pcc[...] = jnp.zeros_like(pcc)
    pso[...] = jnp.zeros_like(pso)
    pco[...] = jnp.zeros_like(pco)

  psc[...] += lax.dot_general(ohc, a, dn, preferred_element_type=jnp.float32)
  pcc[...] += lax.dot_general(ohc, ones, dn,
                              preferred_element_type=jnp.float32)
  pso[...] += lax.dot_general(oho, b, dn, preferred_element_type=jnp.float32)
  pco[...] += lax.dot_general(oho, ones, dn,
                              preferred_element_type=jnp.float32)


def _k5(xc, xo, sc, so, cc, co, bidc, bido, wc, wo):
  g = jax.ShapeDtypeStruct((_NG, _H), jnp.float32)
  wspec = [_full((_H, _H)), _full((1, _H)), _full((_H, 3 * _H)),
           _full((1, 3 * _H)), _full((1, 3 * _H))]
  cspec = pl.BlockSpec((_RB, 1), lambda i: (i, 0))
  bspec = pl.BlockSpec((_RB, 1), lambda i: (i, 0))
  gspec = pl.BlockSpec((_NG, _H), lambda i: (0, 0))
  return pl.pallas_call(
      _k5_body,
      grid=(_GRID,),
      in_specs=[_blk(), _blk(), _blk(), _blk(), cspec, cspec, bspec, bspec]
      + wspec + wspec,
      out_specs=[gspec, gspec, gspec, gspec],
      out_shape=[g, g, g, g],
  )(xc, xo, sc, so, cc, co, bidc, bido, *wc, *wo)


def _k6_body(psc, pcc, pso, pco, linw, linb, outw, outb, out):
  a = psc[...] / jnp.maximum(pcc[...], 1.0)
  b = pso[...] / jnp.maximum(pco[...], 1.0)
  x = jnp.concatenate([a, b], axis=1)
  x = jnp.maximum(_dot(x, linw[...]) + linb[...], 0.0)
  x = jnp.maximum(_dot(x, linw[...]) + linb[...], 0.0)
  out[...] = _dot(x, outw[...]) + outb[...]


def _k6(psc, pcc, pso, pco, linw, linb, outw, outb):
  return pl.pallas_call(
      _k6_body,
      out_shape=jax.ShapeDtypeStruct((_NG, _H), jnp.float32),
  )(psc, pcc, pso, pco, linw, linb, outw, outb)


def _prep_edges(ei):
  src = jnp.pad(ei[0].astype(jnp.int32), (0, _EPAD - _E))
  dst = jnp.pad(ei[1].astype(jnp.int32), (0, _EPAD - _E), constant_values=_N)
  return (src.reshape(_TILES, _NCH, _CHUNK),
          dst.reshape(_TILES, _NCH, _CHUNK))


def kernel(x_constraint, x_operator, edge_index_co, edge_index_oc,
           batch_constraint, batch_operator, params):
  p = params
  xc = jnp.pad(x_constraint, ((0, _NP - _N), (0, 0)))
  xo = jnp.pad(x_operator, ((0, _NP - _N), (0, 0)))
  oc_s, oc_d = _prep_edges(edge_index_oc)
  co_s, co_d = _prep_edges(edge_index_co)
  bidc = jnp.pad(batch_constraint.astype(jnp.int32), (0, _NP - _N),
                 constant_values=_NG).reshape(_NP, 1)
  bido = jnp.pad(batch_operator.astype(jnp.int32), (0, _NP - _N),
                 constant_values=_NG).reshape(_NP, 1)

  def r1(v):
    return v.reshape(1, -1)

  xc0, xo0, tc0, to0 = _k1(
      xc, xo, p['lin_W_constraint'], r1(p['lin_b_constraint']),
      p['lin_W_operator'], r1(p['lin_b_operator']),
      p['W_msg_constraint_0'], p['W_msg_operator_0'])

  sum_c0, sum_o0, cnt_c, cnt_o = _get_sc_kernel(True)(
      to0, tc0, oc_s, oc_d, co_s, co_d)
  cc = cnt_c.reshape(_NP, 1)
  co = cnt_o.reshape(_NP, 1)

  def wpack(t, l, with_msg_next):
    w = [p['W_self_%s_%d' % (t, l)], r1(p['b_conv_%s_%d' % (t, l)]),
         p['gru_Wi_' + t], r1(p['gru_bi_' + t]), r1(p['gru_bh_' + t])]
    if with_msg_next:
      w.append(p['W_msg_%s_%d' % (t, l + 1)])
    return w

  xc1, xo1, tc1, to1 = _k3(xc0, xo0, sum_c0, sum_o0, cc, co,
                           wpack('constraint', 0, True),
                           wpack('operator', 0, True))

  sum_c1, sum_o1 = _get_sc_kernel(False)(to1, tc1, oc_s, oc_d, co_s, co_d)

  psc, pcc, pso, pco = _k5(xc1, xo1, sum_c1, sum_o1, cc, co, bidc, bido,
                           wpack('constraint', 1, False),
                           wpack('operator', 1, False))

  outw = jnp.pad(p['out_W'], ((0, 0), (0, _H - p['out_W'].shape[1])))
  outb = jnp.pad(p['out_b'], (0, _H - p['out_b'].shape[0])).reshape(1, _H)
  out = _k6(psc, pcc, pso, pco, p['lin_W'], r1(p['lin_b']), outw, outb)
  return out[:, :p['out_W'].shape[1]]
